# baseline (speedup 1.0000x reference)
import jax
import jax.numpy as jnp
from jax import lax
from jax.experimental import pallas as pl
from jax.experimental.pallas import tpu as pltpu
from jax.experimental.pallas import tpu_sc as plsc

DIM = 128
NUM_CORES = 2
NUM_SUBCORES = 16
NUM_WORKERS = NUM_CORES * NUM_SUBCORES
STRIPE = 128
NCH = 2
CHUNK = 64
NBUF = 10


def _emb_body(idx_hbm, table_hbm, out_hbm, idx_v, rows_v, gsem, osem):
    n_groups = idx_hbm.shape[1] * (STRIPE // CHUNK)
    n_rot = n_groups // NBUF
    wid = lax.axis_index("s") * NUM_CORES + lax.axis_index("c")
    b0 = wid * STRIPE
    pltpu.sync_copy(idx_hbm.at[wid], idx_v)

    def gstart(g, b):
        s, h = g // NCH, g % NCH
        pltpu.async_copy(table_hbm.at[idx_v.at[s, pl.ds(h * CHUNK, CHUNK)]],
                         rows_v.at[b], gsem.at[b])

    def gwait(g, b):
        s, h = g // NCH, g % NCH
        pltpu.make_async_copy(
            table_hbm.at[idx_v.at[s, pl.ds(h * CHUNK, CHUNK)]],
            rows_v.at[b], gsem.at[b]).wait()

    def wstart(g, b):
        s, h = g // NCH, g % NCH
        pltpu.async_copy(rows_v.at[b],
                         out_hbm.at[s, pl.ds(b0 + h * CHUNK, CHUNK)],
                         osem.at[b])

    def wwait(g, b):
        s, h = g // NCH, g % NCH
        pltpu.make_async_copy(rows_v.at[b],
                              out_hbm.at[s, pl.ds(b0 + h * CHUNK, CHUNK)],
                              osem.at[b]).wait()

    for b in range(NBUF):
        gstart(b, b)

    def body(r, carry):
        g0 = r * NBUF
        for b in range(NBUF):
            gwait(g0 + b, b)
            wstart(g0 + b, b)
        for b in range(NBUF):
            wwait(g0 + b, b)
            gstart(g0 + NBUF + b, b)
        return carry

    lax.fori_loop(0, n_rot - 1, body, 0)

    g0 = (n_rot - 1) * NBUF
    for b in range(NBUF):
        gwait(g0 + b, b)
        wstart(g0 + b, b)
    for b in range(NBUF):
        wwait(g0 + b, b)


def kernel(input_ids, embedding_weight):
    batch, seq = input_ids.shape
    assert batch == NUM_WORKERS * STRIPE
    assert (seq * STRIPE // CHUNK) % NBUF == 0

    idx = (input_ids.astype(jnp.int32)
           .reshape(NUM_WORKERS, STRIPE, seq)
           .transpose(0, 2, 1))
    mesh = plsc.VectorSubcoreMesh(core_axis_name="c", subcore_axis_name="s")
    out = pl.kernel(
        _emb_body,
        out_type=jax.ShapeDtypeStruct((seq, batch, DIM), jnp.float32),
        mesh=mesh,
        scratch_types=[
            pltpu.VMEM((seq, STRIPE), jnp.int32),
            pltpu.VMEM((NBUF, CHUNK, DIM), jnp.float32),
            pltpu.SemaphoreType.DMA((NBUF,)),
            pltpu.SemaphoreType.DMA((NBUF,)),
        ],
    )(idx, embedding_weight)
    return out.transpose(1, 0, 2)
